# baseline (speedup 1.0000x reference)
import functools

import jax
import jax.numpy as jnp
from jax import lax
from jax.experimental import pallas as pl
from jax.experimental.pallas import tpu as pltpu
from jax.experimental.pallas import tpu_sc as plsc

_B = 8
_N = 64
_NN = _N * _N
_OFFS = ((-1, -1), (-1, 0), (-1, 1), (0, -1), (0, 1), (1, -1), (1, 0), (1, 1))


def _shift(a, dy, dx, fill):
    s = a
    if dy:
        s = jnp.roll(s, -dy, axis=1)
    if dx:
        s = jnp.roll(s, -dx, axis=2)
    yi = lax.broadcasted_iota(jnp.int32, (_B, _N, _N), 1)
    xi = lax.broadcasted_iota(jnp.int32, (_B, _N, _N), 2)
    valid = (yi + dy >= 0) & (yi + dy <= _N - 1) & (xi + dx >= 0) & (xi + dx <= _N - 1)
    return jnp.where(valid, s, fill)


def _dist_pred_kernel(w_ref, pred_ref):
    w = w_ref[...]
    inf = jnp.float32(jnp.inf)
    flat = (lax.broadcasted_iota(jnp.int32, (_B, _N, _N), 1) * _N
            + lax.broadcasted_iota(jnp.int32, (_B, _N, _N), 2))
    dist0 = jnp.where(flat == 0, jnp.float32(0.0), inf)

    def sweep(dist):
        best = jnp.full((_B, _N, _N), inf, jnp.float32)
        for dy, dx in _OFFS:
            best = jnp.minimum(best, _shift(dist, dy, dx, inf))
        return jnp.minimum(dist, best + w)

    def bf_cond(c):
        _, changed, it = c
        return jnp.logical_and(changed, it < _NN)

    def bf_body(c):
        dist, _, it = c
        nd = sweep(dist)
        changed = jnp.any(nd < dist)
        return (nd, changed, it + 1)

    dist, _, _ = lax.while_loop(bf_cond, bf_body,
                                (dist0, jnp.bool_(True), jnp.int32(0)))

    best = jnp.full((_B, _N, _N), inf, jnp.float32)
    off = jnp.zeros((_B, _N, _N), jnp.int32)
    for dy, dx in _OFFS:
        nd = _shift(dist, dy, dx, inf)
        take = nd < best
        off = jnp.where(take, jnp.int32(dy * _N + dx), off)
        best = jnp.minimum(best, nd)
    pred_ref[...] = jnp.where(flat == 0, 0, flat + off).reshape(_B, _NN)


def _backtrack_tec(pred_hbm, path_hbm, pred_v, path_v):
    wid = lax.axis_index("s") * 2 + lax.axis_index("c")

    @pl.when(wid < _B)
    def _():
        b = wid
        pltpu.sync_copy(pred_hbm.at[b], pred_v.at[pl.ds(0, _NN)])

        zeros16 = jnp.zeros((16,), jnp.float32)

        def zbody(i, _):
            path_v[pl.ds(i * 16, 16)] = zeros16
            return 0

        lax.fori_loop(0, _NN // 16, zbody, 0)

        lanes = lax.iota(jnp.int32, 16)

        def mark(cell):
            vec = path_v[pl.ds(cell, 16)]
            path_v[pl.ds(cell, 16)] = jnp.where(lanes == 0,
                                                jnp.float32(1.0), vec)

        mark(jnp.int32(_NN - 1))

        def bt_body(i, cur):
            nxt = pred_v[pl.ds(cur, 16)][0]
            mark(nxt)
            return nxt

        lax.fori_loop(0, _NN, bt_body, jnp.int32(_NN - 1))
        pltpu.sync_copy(path_v.at[pl.ds(0, _NN)], path_hbm.at[b])


@functools.cache
def _backtrack_sc():
    return pl.kernel(
        _backtrack_tec,
        out_type=jax.ShapeDtypeStruct((_B, _NN), jnp.float32),
        mesh=plsc.VectorSubcoreMesh(core_axis_name="c", subcore_axis_name="s"),
        scratch_types=[
            pltpu.VMEM((_NN + 16,), jnp.int32),
            pltpu.VMEM((_NN + 16,), jnp.float32),
        ],
    )


def kernel(weights):
    pred = pl.pallas_call(
        _dist_pred_kernel,
        out_shape=jax.ShapeDtypeStruct((_B, _NN), jnp.int32),
    )(weights)
    path = _backtrack_sc()(pred)
    return path.reshape(_B, _N, _N)


# speedup vs baseline: 1.4705x; 1.4705x over previous
import functools

import jax
import jax.numpy as jnp
from jax import lax
from jax.experimental import pallas as pl
from jax.experimental.pallas import tpu as pltpu
from jax.experimental.pallas import tpu_sc as plsc

_B = 8
_N = 64
_NN = _N * _N
_OFFS = ((-1, -1), (-1, 0), (-1, 1), (0, -1), (0, 1), (1, -1), (1, 0), (1, 1))


def _shift(a, dy, dx, fill):
    s = a
    if dy:
        s = jnp.roll(s, -dy, axis=1)
    if dx:
        s = jnp.roll(s, -dx, axis=2)
    yi = lax.broadcasted_iota(jnp.int32, (_B, _N, _N), 1)
    xi = lax.broadcasted_iota(jnp.int32, (_B, _N, _N), 2)
    valid = (yi + dy >= 0) & (yi + dy <= _N - 1) & (xi + dx >= 0) & (xi + dx <= _N - 1)
    return jnp.where(valid, s, fill)


def _dist_pred_kernel(w_ref, pred_ref):
    w = w_ref[...]
    inf = jnp.float32(jnp.inf)
    flat = (lax.broadcasted_iota(jnp.int32, (_B, _N, _N), 1) * _N
            + lax.broadcasted_iota(jnp.int32, (_B, _N, _N), 2))
    dist0 = jnp.where(flat == 0, jnp.float32(0.0), inf)

    def sweep(dist):
        best = jnp.full((_B, _N, _N), inf, jnp.float32)
        for dy, dx in _OFFS:
            best = jnp.minimum(best, _shift(dist, dy, dx, inf))
        return jnp.minimum(dist, best + w)

    def bf_cond(c):
        _, changed, it = c
        return jnp.logical_and(changed, it < _NN)

    def bf_body(c):
        dist, _, it = c
        nd = sweep(dist)
        changed = jnp.any(nd < dist)
        return (nd, changed, it + 1)

    dist, _, _ = lax.while_loop(bf_cond, bf_body,
                                (dist0, jnp.bool_(True), jnp.int32(0)))

    best = jnp.full((_B, _N, _N), inf, jnp.float32)
    off = jnp.zeros((_B, _N, _N), jnp.int32)
    for dy, dx in _OFFS:
        nd = _shift(dist, dy, dx, inf)
        take = nd < best
        off = jnp.where(take, jnp.int32(dy * _N + dx), off)
        best = jnp.minimum(best, nd)
    pred_ref[...] = jnp.where(flat == 0, 0, flat + off).reshape(_B, _NN)


def _backtrack_tec(pred_hbm, path_hbm, pred_v, path_v, cur_s):
    wid = lax.axis_index("s") * 2 + lax.axis_index("c")

    @pl.when(wid < _B)
    def _():
        b = wid
        pltpu.sync_copy(pred_hbm.at[b], pred_v.at[pl.ds(0, _NN)])

        zeros16 = jnp.zeros((16,), jnp.float32)

        def zbody(i, _):
            path_v[pl.ds(i * 16, 16)] = zeros16
            return 0

        lax.fori_loop(0, _NN // 16, zbody, 0)

        lanes = lax.iota(jnp.int32, 16)

        def mark(cell):
            vec = path_v[pl.ds(cell, 16)]
            path_v[pl.ds(cell, 16)] = jnp.where(lanes == 0,
                                                jnp.float32(1.0), vec)

        mark(jnp.int32(_NN - 1))

        cur_s[0] = jnp.int32(_NN - 1)

        def chunk(i, t):
            c = cur_s[0]

            @pl.when(c != 0)
            def _():
                def step(j, cur):
                    nxt = pred_v[pl.ds(cur, 16)][0]
                    mark(nxt)
                    return nxt

                cur_s[0] = lax.fori_loop(0, 128, step, c)

            return t

        lax.fori_loop(0, _NN // 128, chunk, 0)
        pltpu.sync_copy(path_v.at[pl.ds(0, _NN)], path_hbm.at[b])


@functools.cache
def _backtrack_sc():
    return pl.kernel(
        _backtrack_tec,
        out_type=jax.ShapeDtypeStruct((_B, _NN), jnp.float32),
        mesh=plsc.VectorSubcoreMesh(core_axis_name="c", subcore_axis_name="s"),
        scratch_types=[
            pltpu.VMEM((_NN + 16,), jnp.int32),
            pltpu.VMEM((_NN + 16,), jnp.float32),
            pltpu.SMEM((1,), jnp.int32),
        ],
    )


def kernel(weights):
    pred = pl.pallas_call(
        _dist_pred_kernel,
        out_shape=jax.ShapeDtypeStruct((_B, _NN), jnp.int32),
    )(weights)
    path = _backtrack_sc()(pred)
    return path.reshape(_B, _N, _N)


# speedup vs baseline: 4.3886x; 2.9845x over previous
import functools

import jax
import jax.numpy as jnp
from jax import lax
from jax.experimental import pallas as pl
from jax.experimental.pallas import tpu as pltpu
from jax.experimental.pallas import tpu_sc as plsc

_B = 8
_N = 64
_NN = _N * _N
_P = _B // 2
_XL = 2 * _N
_OFFS = ((-1, -1), (-1, 0), (-1, 1), (0, -1), (0, 1), (1, -1), (1, 0), (1, 1))


def _dist_pred_kernel(w_ref, pred_ref):
    w = w_ref[...]
    inf = jnp.float32(jnp.inf)
    shape = (_P, _N, _XL)
    yi = lax.broadcasted_iota(jnp.int32, shape, 1)
    xi = lax.broadcasted_iota(jnp.int32, shape, 2)
    xm = xi & (_N - 1)
    flat = yi * _N + xm
    take_xp = xm != _N - 1
    take_xn = xm != 0
    take_yp = yi != _N - 1
    take_yn = yi != 0

    dist0 = jnp.where(flat == 0, jnp.float32(0.0), inf)

    def box9(d):
        h = jnp.minimum(d, jnp.where(take_xp, jnp.roll(d, -1, axis=2), inf))
        h = jnp.minimum(h, jnp.where(take_xn, jnp.roll(d, 1, axis=2), inf))
        v = jnp.minimum(h, jnp.where(take_yp, jnp.roll(h, -1, axis=1), inf))
        v = jnp.minimum(v, jnp.where(take_yn, jnp.roll(h, 1, axis=1), inf))
        return v

    def bf_cond(c):
        _, changed, it = c
        return jnp.logical_and(changed, it < _NN)

    def bf_body(c):
        dist, _, it = c
        nd = dist
        for _u in range(8):
            nd = jnp.minimum(nd, box9(nd) + w)
        changed = jnp.any(nd < dist)
        return (nd, changed, it + 8)

    dist, _, _ = lax.while_loop(bf_cond, bf_body,
                                (dist0, jnp.bool_(True), jnp.int32(0)))

    best = jnp.full(shape, inf, jnp.float32)
    off = jnp.zeros(shape, jnp.int32)
    for dy, dx in _OFFS:
        s = dist
        if dy:
            s = jnp.roll(s, -dy, axis=1)
        if dx:
            s = jnp.roll(s, -dx, axis=2)
        valid = ((yi + dy >= 0) & (yi + dy <= _N - 1)
                 & (xm + dx >= 0) & (xm + dx <= _N - 1))
        nd = jnp.where(valid, s, inf)
        take = nd < best
        off = jnp.where(take, jnp.int32(dy * _N + dx), off)
        best = jnp.minimum(best, nd)
    pred_ref[...] = jnp.where(flat == 0, 0, flat + off)


def _backtrack_tec(pred_hbm, path_hbm, pred_v, path_v, cur_s):
    wid = lax.axis_index("s") * 2 + lax.axis_index("c")

    @pl.when(wid < _B)
    def _():
        b = wid
        pltpu.sync_copy(pred_hbm.at[b], pred_v.at[pl.ds(0, _NN)])

        zeros16 = jnp.zeros((16,), jnp.float32)

        def zbody(i, _):
            path_v[pl.ds(i * 16, 16)] = zeros16
            return 0

        lax.fori_loop(0, _NN // 16, zbody, 0)

        lanes = lax.iota(jnp.int32, 16)

        def mark(cell):
            vec = path_v[pl.ds(cell, 16)]
            path_v[pl.ds(cell, 16)] = jnp.where(lanes == 0,
                                                jnp.float32(1.0), vec)

        mark(jnp.int32(_NN - 1))

        cur_s[0] = jnp.int32(_NN - 1)

        def chunk(i, t):
            c = cur_s[0]

            @pl.when(c != 0)
            def _():
                def step(j, cur):
                    nxt = pred_v[pl.ds(cur, 16)][0]
                    mark(nxt)
                    return nxt

                cur_s[0] = lax.fori_loop(0, 128, step, c)

            return t

        lax.fori_loop(0, _NN // 128, chunk, 0)
        pltpu.sync_copy(path_v.at[pl.ds(0, _NN)], path_hbm.at[b])


@functools.cache
def _backtrack_sc():
    return pl.kernel(
        _backtrack_tec,
        out_type=jax.ShapeDtypeStruct((_B, _NN), jnp.float32),
        mesh=plsc.VectorSubcoreMesh(core_axis_name="c", subcore_axis_name="s"),
        scratch_types=[
            pltpu.VMEM((_NN + 16,), jnp.int32),
            pltpu.VMEM((_NN + 16,), jnp.float32),
            pltpu.SMEM((1,), jnp.int32),
        ],
    )


def kernel(weights):
    w2 = weights.reshape(_P, 2, _N, _N).transpose(0, 2, 1, 3).reshape(
        _P, _N, _XL)
    pred2 = pl.pallas_call(
        _dist_pred_kernel,
        out_shape=jax.ShapeDtypeStruct((_P, _N, _XL), jnp.int32),
    )(w2)
    pred = pred2.reshape(_P, _N, 2, _N).transpose(0, 2, 1, 3).reshape(_B, _NN)
    path = _backtrack_sc()(pred)
    return path.reshape(_B, _N, _N)


# speedup vs baseline: 4.5962x; 1.0473x over previous
import functools

import jax
import jax.numpy as jnp
from jax import lax
from jax.experimental import pallas as pl
from jax.experimental.pallas import tpu as pltpu
from jax.experimental.pallas import tpu_sc as plsc

_B = 8
_N = 64
_NN = _N * _N
_P = _B // 2
_XL = 2 * _N
_OFFS = ((-1, -1), (-1, 0), (-1, 1), (0, -1), (0, 1), (1, -1), (1, 0), (1, 1))


def _dist_pred_kernel(w_ref, pred_ref):
    w8 = w_ref[...].reshape(_P, 2, _N, _N)
    w = jnp.concatenate([w8[:, 0], w8[:, 1]], axis=2)
    inf = jnp.float32(jnp.inf)
    shape = (_P, _N, _XL)
    yi = lax.broadcasted_iota(jnp.int32, shape, 1)
    xi = lax.broadcasted_iota(jnp.int32, shape, 2)
    xm = xi & (_N - 1)
    flat = yi * _N + xm
    take_xp = xm != _N - 1
    take_xn = xm != 0
    take_yp = yi != _N - 1
    take_yn = yi != 0

    dist0 = jnp.where(flat == 0, jnp.float32(0.0), inf)

    def box9(d):
        h = jnp.minimum(d, jnp.where(take_xp, jnp.roll(d, -1, axis=2), inf))
        h = jnp.minimum(h, jnp.where(take_xn, jnp.roll(d, 1, axis=2), inf))
        v = jnp.minimum(h, jnp.where(take_yp, jnp.roll(h, -1, axis=1), inf))
        v = jnp.minimum(v, jnp.where(take_yn, jnp.roll(h, 1, axis=1), inf))
        return v

    def bf_cond(c):
        _, changed, it = c
        return jnp.logical_and(changed, it < _NN)

    def bf_body(c):
        dist, _, it = c
        nd = dist
        for _u in range(8):
            nd = jnp.minimum(nd, box9(nd) + w)
        changed = jnp.any(nd < dist)
        return (nd, changed, it + 8)

    dist, _, _ = lax.while_loop(bf_cond, bf_body,
                                (dist0, jnp.bool_(True), jnp.int32(0)))

    best = jnp.full(shape, inf, jnp.float32)
    off = jnp.zeros(shape, jnp.int32)
    for dy, dx in _OFFS:
        s = dist
        if dy:
            s = jnp.roll(s, -dy, axis=1)
        if dx:
            s = jnp.roll(s, -dx, axis=2)
        valid = ((yi + dy >= 0) & (yi + dy <= _N - 1)
                 & (xm + dx >= 0) & (xm + dx <= _N - 1))
        nd = jnp.where(valid, s, inf)
        take = nd < best
        off = jnp.where(take, jnp.int32(dy * _N + dx), off)
        best = jnp.minimum(best, nd)
    packed = jnp.where(flat == 0, 0, flat + off)
    pred_ref[...] = jnp.stack(
        [packed[:, :, :_N], packed[:, :, _N:]], axis=1).reshape(_B, _N, _N)


def _backtrack_tec(pred_hbm, path_hbm, pred_v, path_v, cur_s):
    wid = lax.axis_index("s") * 2 + lax.axis_index("c")

    @pl.when(wid < _B)
    def _():
        b = wid
        pltpu.sync_copy(pred_hbm.at[b], pred_v.at[pl.ds(0, _NN)])

        zeros16 = jnp.zeros((16,), jnp.float32)

        def zbody(i, _):
            path_v[pl.ds(i * 16, 16)] = zeros16
            return 0

        lax.fori_loop(0, _NN // 16, zbody, 0)

        lanes = lax.iota(jnp.int32, 16)

        def mark(cell):
            vec = path_v[pl.ds(cell, 16)]
            path_v[pl.ds(cell, 16)] = jnp.where(lanes == 0,
                                                jnp.float32(1.0), vec)

        mark(jnp.int32(_NN - 1))

        cur_s[0] = jnp.int32(_NN - 1)

        def chunk(i, t):
            c = cur_s[0]

            @pl.when(c != 0)
            def _():
                def step(j, cur):
                    nxt = pred_v[pl.ds(cur, 16)][0]
                    mark(nxt)
                    return nxt

                cur_s[0] = lax.fori_loop(0, 128, step, c)

            return t

        lax.fori_loop(0, _NN // 128, chunk, 0)
        pltpu.sync_copy(path_v.at[pl.ds(0, _NN)], path_hbm.at[b])


@functools.cache
def _backtrack_sc():
    return pl.kernel(
        _backtrack_tec,
        out_type=jax.ShapeDtypeStruct((_B, _NN), jnp.float32),
        mesh=plsc.VectorSubcoreMesh(core_axis_name="c", subcore_axis_name="s"),
        scratch_types=[
            pltpu.VMEM((_NN + 16,), jnp.int32),
            pltpu.VMEM((_NN + 16,), jnp.float32),
            pltpu.SMEM((1,), jnp.int32),
        ],
    )


def kernel(weights):
    pred = pl.pallas_call(
        _dist_pred_kernel,
        out_shape=jax.ShapeDtypeStruct((_B, _N, _N), jnp.int32),
    )(weights)
    path = _backtrack_sc()(pred.reshape(_B, _NN))
    return path.reshape(_B, _N, _N)
